# Initial kernel scaffold; baseline (speedup 1.0000x reference)
import jax
import jax.numpy as jnp
from jax.experimental import pallas as pl


def kernel(y, W1, b1, W2, b2, W3, b3):
    raise NotImplementedError("write your pallas kernel here")


# speedup vs baseline: 618.5385x; 618.5385x over previous
import jax
import jax.numpy as jnp
from jax.experimental import pallas as pl
from jax.experimental.pallas import tpu as pltpu

_N_STATES = 16
_T = 2048
_B = 512
_TCK = 16
_NCHUNK = _T // _TCK
_NC = _TCK * _B


def _viterbi_kernel(y_ref, w1_ref, b1_ref, w2_ref, b2_ref, w3_ref, b3_ref,
                    out_ref, dec_ref):
    w1 = w1_ref[:]
    b1 = b1_ref[:]
    w2 = w2_ref[:]
    b2 = b2_ref[:]
    w3 = w3_ref[:]
    b3 = b3_ref[:]
    pow2 = 1 << jax.lax.broadcasted_iota(jnp.int32, (8, 1), 0)

    def fwd_chunk(c, carry):
        y_c = y_ref[pl.ds(c, 1), :]
        h1 = 1.0 / (1.0 + jnp.exp(-(w1 * y_c + b1)))
        h2 = jnp.maximum(
            jnp.dot(w2, h1, preferred_element_type=jnp.float32) + b2, 0.0)
        pri = jnp.dot(w3, h2, preferred_element_type=jnp.float32) + b3
        rows = []
        for t in range(_TCK):
            pri_t = pri[:, t * _B:(t + 1) * _B]
            lo = carry[0:8, :]
            hi = carry[8:16, :]
            m8 = jnp.minimum(lo, hi)
            b8 = (hi < lo).astype(jnp.int32)
            carry = pri_t + jnp.repeat(m8, 2, axis=0)
            rows.append(jnp.sum(b8 * pow2, axis=0, keepdims=True))
        dec_ref[pl.ds(c * _TCK, _TCK), :] = jnp.concatenate(rows, axis=0)
        return carry

    carry0 = jnp.zeros((_N_STATES, _B), jnp.float32)
    jax.lax.fori_loop(0, _NCHUNK, fwd_chunk, carry0)

    def bwd_chunk(i, state):
        c = _NCHUNK - 1 - i
        dblk = dec_ref[pl.ds(c * _TCK, _TCK), :]
        outs = [None] * _TCK
        for t in range(_TCK - 1, -1, -1):
            d_t = dblk[t:t + 1, :]
            p = jax.lax.shift_right_logical(state, 1)
            bit = jax.lax.shift_right_logical(d_t, p) & 1
            state = p + (bit << 3)
            outs[t] = bit
        out_ref[pl.ds(c * _TCK, _TCK), :] = (
            jnp.concatenate(outs, axis=0).astype(jnp.float32))
        return state

    state0 = jnp.zeros((1, _B), jnp.int32)
    jax.lax.fori_loop(0, _NCHUNK, bwd_chunk, state0)


def _decode(y, W1, b1, W2, b2, W3, b3, interpret=False):
    y_lin = y.T.reshape(_NCHUNK, _NC)
    out = pl.pallas_call(
        _viterbi_kernel,
        out_shape=jax.ShapeDtypeStruct((_T, _B), jnp.float32),
        scratch_shapes=[pltpu.VMEM((_T, _B), jnp.int32)],
        interpret=interpret,
    )(y_lin,
      W1.T.reshape(100, 1), b1.reshape(100, 1),
      W2.T, b2.reshape(50, 1),
      W3.T, b3.reshape(16, 1))
    return out.T


@jax.jit
def kernel(y, W1, b1, W2, b2, W3, b3):
    return _decode(y, W1, b1, W2, b2, W3, b3)


# speedup vs baseline: 815.2857x; 1.3181x over previous
import jax
import jax.numpy as jnp
from jax.experimental import pallas as pl
from jax.experimental.pallas import tpu as pltpu

_N_STATES = 16
_T = 2048
_B = 512
_TCK = 16
_NCHUNK = _T // _TCK
_NC = _TCK * _B


def _viterbi_kernel(y_ref, w1_ref, b1_ref, w2_ref, b2_ref, w3_ref, b3_ref,
                    out_ref, dec_ref):
    w1 = w1_ref[:]
    b1 = b1_ref[:]
    w2 = w2_ref[:]
    b2 = b2_ref[:]
    w3 = w3_ref[:]
    b3 = b3_ref[:]
    pow2 = 1 << jax.lax.broadcasted_iota(jnp.int32, (8, 1), 0)

    def fwd_chunk(c, carry):
        y_c = y_ref[pl.ds(c, 1), :]
        h1 = jnp.tanh(w1 * y_c + b1)
        h2 = jnp.maximum(
            jnp.dot(w2, h1, preferred_element_type=jnp.float32) + b2, 0.0)
        pri = jnp.dot(w3, h2, preferred_element_type=jnp.float32) + b3
        rows = []
        for t in range(_TCK):
            pri_t = pri[:, t * _B:(t + 1) * _B]
            lo = carry[0:8, :]
            hi = carry[8:16, :]
            m8 = jnp.minimum(lo, hi)
            b8 = (hi < lo).astype(jnp.int32)
            m16 = jnp.concatenate(
                [m8.reshape(8, 1, _B)] * 2, axis=1).reshape(16, _B)
            carry = pri_t + m16
            rows.append(jnp.sum(b8 * pow2, axis=0, keepdims=True))
        dec_ref[pl.ds(c * _TCK, _TCK), :] = jnp.concatenate(rows, axis=0)
        return carry

    carry0 = jnp.zeros((_N_STATES, _B), jnp.float32)
    jax.lax.fori_loop(0, _NCHUNK, fwd_chunk, carry0)

    def bwd_chunk(i, state):
        c = _NCHUNK - 1 - i
        dblk = dec_ref[pl.ds(c * _TCK, _TCK), :]
        outs = [None] * _TCK
        for t in range(_TCK - 1, -1, -1):
            d_t = dblk[t:t + 1, :]
            p = jax.lax.shift_right_logical(state, 1)
            bit = jax.lax.shift_right_logical(d_t, p) & 1
            state = p + (bit << 3)
            outs[t] = bit
        out_ref[pl.ds(c * _TCK, _TCK), :] = (
            jnp.concatenate(outs, axis=0).astype(jnp.float32))
        return state

    state0 = jnp.zeros((1, _B), jnp.int32)
    jax.lax.fori_loop(0, _NCHUNK, bwd_chunk, state0)


def _decode(y, W1, b1, W2, b2, W3, b3, interpret=False):
    y_lin = y.T.reshape(_NCHUNK, _NC)
    w1h = 0.5 * W1.T.reshape(100, 1)
    b1h = 0.5 * b1.reshape(100, 1)
    w2h = 0.5 * W2.T
    b2h = (b2 + 0.5 * W2.sum(axis=0)).reshape(50, 1)
    out = pl.pallas_call(
        _viterbi_kernel,
        out_shape=jax.ShapeDtypeStruct((_T, _B), jnp.float32),
        scratch_shapes=[pltpu.VMEM((_T, _B), jnp.int32)],
        interpret=interpret,
    )(y_lin, w1h, b1h, w2h, b2h, W3.T, b3.reshape(16, 1))
    return out.T


@jax.jit
def kernel(y, W1, b1, W2, b2, W3, b3):
    return _decode(y, W1, b1, W2, b2, W3, b3)
